# Initial kernel scaffold; baseline (speedup 1.0000x reference)
import jax
import jax.numpy as jnp
from jax.experimental import pallas as pl


def kernel(inputs, token_table, pos_table):
    raise NotImplementedError("write your pallas kernel here")


# speedup vs baseline: 3.9541x; 3.9541x over previous
import jax
import jax.numpy as jnp
from jax import lax
from jax.experimental import pallas as pl
from jax.experimental.pallas import tpu as pltpu
from jax.experimental.pallas import tpu_sc as plsc

_LANES = 16
_IDXCOLS = 100


def _emb_body(idx_hbm, tok_hbm, pos_hbm, out_hbm, pos_v, idx_v, rows_v, sem):
    seq = pos_v.shape[0]
    d = pos_v.shape[1]
    n_workers = 32
    nseq_total = out_hbm.shape[0] // seq
    nseq_per_w = nseq_total // n_workers
    wid = lax.axis_index("s") * 2 + lax.axis_index("c")

    pltpu.sync_copy(pos_hbm, pos_v)

    def seq_body(c, carry):
        g = wid * nseq_per_w + c
        pltpu.sync_copy(idx_hbm.at[pl.ds(2 * g, 2)], idx_v)
        cp0 = pltpu.async_copy(
            tok_hbm.at[idx_v.at[0]], rows_v.at[pl.ds(0, _IDXCOLS)], sem)
        cp1 = pltpu.async_copy(
            tok_hbm.at[idx_v.at[1]], rows_v.at[pl.ds(_IDXCOLS, _IDXCOLS)], sem)
        cp0.wait()
        cp1.wait()

        def add_body(r, acc):
            for j in range(d // _LANES):
                sl = pl.ds(j * _LANES, _LANES)
                rows_v[r, sl] = rows_v[r, sl] + pos_v[r, sl]
            return acc

        lax.fori_loop(0, seq, add_body, 0)
        pltpu.sync_copy(rows_v, out_hbm.at[pl.ds(g * seq, seq)])
        return carry

    lax.fori_loop(0, nseq_per_w, seq_body, 0)


def kernel(inputs, token_table, pos_table):
    b, s = inputs.shape
    v, d = token_table.shape
    idx = inputs.astype(jnp.int32).reshape(b * s // _IDXCOLS, _IDXCOLS)
    mesh = plsc.VectorSubcoreMesh(core_axis_name="c", subcore_axis_name="s")
    run = pl.kernel(
        _emb_body,
        mesh=mesh,
        out_type=jax.ShapeDtypeStruct((b * s, d), jnp.float32),
        scratch_types=[
            pltpu.VMEM((s, d), jnp.float32),
            pltpu.VMEM((2, _IDXCOLS), jnp.int32),
            pltpu.VMEM((s, d), jnp.float32),
            pltpu.SemaphoreType.DMA,
        ],
    )
    out = run(idx, token_table, pos_table)
    return out.reshape(b, s, d)


# speedup vs baseline: 6.4646x; 1.6349x over previous
import jax
import jax.numpy as jnp
from jax import lax
from jax.experimental import pallas as pl
from jax.experimental.pallas import tpu as pltpu
from jax.experimental.pallas import tpu_sc as plsc

_LANES = 16
_IDXCOLS = 100
_NW = 32


def _emb_body(idx_hbm, tok_hbm, pos_hbm, out_hbm,
              pos_v, idx_v, rows0, rows1, gs0, gs1, os0, os1):
    seq, d = pos_v.shape
    nseq = out_hbm.shape[0] // seq // _NW
    wid = lax.axis_index("s") * 2 + lax.axis_index("c")
    seq0 = wid * nseq

    pltpu.sync_copy(pos_hbm, pos_v)
    pltpu.sync_copy(idx_hbm.at[pl.ds(seq0 * 2, nseq * 2)], idx_v)

    rows = (rows0, rows1)
    gs = (gs0, gs1)
    os = (os0, os1)

    def fire_gather(c, b):
        pltpu.async_copy(
            tok_hbm.at[idx_v.at[2 * c]], rows[b].at[pl.ds(0, _IDXCOLS)], gs[b])
        pltpu.async_copy(
            tok_hbm.at[idx_v.at[2 * c + 1]],
            rows[b].at[pl.ds(_IDXCOLS, _IDXCOLS)], gs[b])

    def wait_gather(b):
        pltpu.make_async_copy(tok_hbm.at[pl.ds(0, seq)], rows[b], gs[b]).wait()

    def fire_out(c, b):
        pltpu.async_copy(rows[b], out_hbm.at[pl.ds((seq0 + c) * seq, seq)], os[b])

    def wait_out(b):
        pltpu.make_async_copy(rows[b], out_hbm.at[pl.ds(0, seq)], os[b]).wait()

    def add_pos(b):
        buf = rows[b]

        def add_row(r, acc):
            for j in range(d // _LANES):
                sl = pl.ds(j * _LANES, _LANES)
                buf[r, sl] = buf[r, sl] + pos_v[r, sl]
            return acc

        lax.fori_loop(0, seq, add_row, 0)

    fire_gather(0, 0)

    wait_gather(0)
    fire_gather(1, 1)
    add_pos(0)
    fire_out(0, 0)

    wait_gather(1)
    wait_out(0)
    fire_gather(2, 0)
    add_pos(1)
    fire_out(1, 1)

    def pair_body(gg, carry):
        c = 2 * gg
        wait_gather(0)
        wait_out(1)
        fire_gather(c + 1, 1)
        add_pos(0)
        fire_out(c, 0)

        wait_gather(1)
        wait_out(0)
        fire_gather(c + 2, 0)
        add_pos(1)
        fire_out(c + 1, 1)
        return carry

    lax.fori_loop(1, nseq // 2 - 1, pair_body, 0)

    c = nseq - 2
    wait_gather(0)
    wait_out(1)
    fire_gather(c + 1, 1)
    add_pos(0)
    fire_out(c, 0)

    wait_gather(1)
    add_pos(1)
    fire_out(c + 1, 1)

    wait_out(0)
    wait_out(1)


def kernel(inputs, token_table, pos_table):
    b, s = inputs.shape
    v, d = token_table.shape
    nseq = b // _NW
    idx = inputs.astype(jnp.int32).reshape(b * s // _IDXCOLS, _IDXCOLS)
    mesh = plsc.VectorSubcoreMesh(core_axis_name="c", subcore_axis_name="s")
    run = pl.kernel(
        _emb_body,
        mesh=mesh,
        out_type=jax.ShapeDtypeStruct((b * s, d), jnp.float32),
        scratch_types=[
            pltpu.VMEM((s, d), jnp.float32),
            pltpu.VMEM((nseq * 2, _IDXCOLS), jnp.int32),
            pltpu.VMEM((s, d), jnp.float32),
            pltpu.VMEM((s, d), jnp.float32),
            pltpu.SemaphoreType.DMA,
            pltpu.SemaphoreType.DMA,
            pltpu.SemaphoreType.DMA,
            pltpu.SemaphoreType.DMA,
        ],
    )
    out = run(idx, token_table, pos_table)
    return out.reshape(b, s, d)


# speedup vs baseline: 6.4754x; 1.0017x over previous
import jax
import jax.numpy as jnp
from jax import lax
from jax.experimental import pallas as pl
from jax.experimental.pallas import tpu as pltpu
from jax.experimental.pallas import tpu_sc as plsc

_LANES = 16
_IDXCOLS = 100
_NW = 32


def _emb_body(idx_hbm, tok_hbm, pos_hbm, out_hbm,
              pos_v, idx_v, rows0, rows1, gs0, gs1, os0, os1):
    seq, d = pos_v.shape
    nseq = out_hbm.shape[0] // seq // _NW
    wid = lax.axis_index("s") * 2 + lax.axis_index("c")
    seq0 = wid * nseq

    pltpu.sync_copy(pos_hbm, pos_v)
    pltpu.sync_copy(idx_hbm.at[pl.ds(seq0 * 2, nseq * 2)], idx_v)

    rows = (rows0, rows1)
    gs = (gs0, gs1)
    os = (os0, os1)

    def fire_gather(c, b):
        pltpu.async_copy(
            tok_hbm.at[idx_v.at[2 * c]], rows[b].at[pl.ds(0, _IDXCOLS)], gs[b])
        pltpu.async_copy(
            tok_hbm.at[idx_v.at[2 * c + 1]],
            rows[b].at[pl.ds(_IDXCOLS, _IDXCOLS)], gs[b])

    def wait_gather(b):
        pltpu.make_async_copy(tok_hbm.at[pl.ds(0, seq)], rows[b], gs[b]).wait()

    def fire_out(c, b):
        pltpu.async_copy(rows[b], out_hbm.at[pl.ds((seq0 + c) * seq, seq)], os[b])

    def wait_out(b):
        pltpu.make_async_copy(rows[b], out_hbm.at[pl.ds(0, seq)], os[b]).wait()

    def add_pos(b):
        buf = rows[b]

        def add_row(r, acc):
            for j in range(d // _LANES):
                sl = pl.ds(j * _LANES, _LANES)
                plsc.addupdate(buf.at[r, sl], pos_v[r, sl])
            return acc

        lax.fori_loop(0, seq, add_row, 0)

    fire_gather(0, 0)

    wait_gather(0)
    fire_gather(1, 1)
    add_pos(0)
    fire_out(0, 0)

    wait_gather(1)
    wait_out(0)
    fire_gather(2, 0)
    add_pos(1)
    fire_out(1, 1)

    def pair_body(gg, carry):
        c = 2 * gg
        wait_gather(0)
        wait_out(1)
        fire_gather(c + 1, 1)
        add_pos(0)
        fire_out(c, 0)

        wait_gather(1)
        wait_out(0)
        fire_gather(c + 2, 0)
        add_pos(1)
        fire_out(c + 1, 1)
        return carry

    lax.fori_loop(1, nseq // 2 - 1, pair_body, 0)

    c = nseq - 2
    wait_gather(0)
    wait_out(1)
    fire_gather(c + 1, 1)
    add_pos(0)
    fire_out(c, 0)

    wait_gather(1)
    add_pos(1)
    fire_out(c + 1, 1)

    wait_out(0)
    wait_out(1)


def kernel(inputs, token_table, pos_table):
    b, s = inputs.shape
    v, d = token_table.shape
    nseq = b // _NW
    idx = inputs.astype(jnp.int32).reshape(b * s // _IDXCOLS, _IDXCOLS)
    mesh = plsc.VectorSubcoreMesh(core_axis_name="c", subcore_axis_name="s")
    run = pl.kernel(
        _emb_body,
        mesh=mesh,
        out_type=jax.ShapeDtypeStruct((b * s, d), jnp.float32),
        scratch_types=[
            pltpu.VMEM((s, d), jnp.float32),
            pltpu.VMEM((nseq * 2, _IDXCOLS), jnp.int32),
            pltpu.VMEM((s, d), jnp.float32),
            pltpu.VMEM((s, d), jnp.float32),
            pltpu.SemaphoreType.DMA,
            pltpu.SemaphoreType.DMA,
            pltpu.SemaphoreType.DMA,
            pltpu.SemaphoreType.DMA,
        ],
    )
    out = run(idx, token_table, pos_table)
    return out.reshape(b, s, d)
